# Initial kernel scaffold; baseline (speedup 1.0000x reference)
import jax
import jax.numpy as jnp
from jax.experimental import pallas as pl


def kernel(x, pos, Wn, bn, W1, b1, W2, b2, Wq, bq, Wk, bk, Wv, bv, Wo, bo, Wp1, bp1, Wp2, bp2):
    raise NotImplementedError("write your pallas kernel here")


# speedup vs baseline: 1.5596x; 1.5596x over previous
import functools

import numpy as np
import jax
import jax.numpy as jnp
from jax import lax
from jax.experimental import pallas as pl
from jax.experimental.pallas import tpu as pltpu
from jax.experimental.pallas import tpu_sc as plsc

N_ATOMS = 100000
K_SEL = 1000
NODE_F = 8
HIDDEN = 128
OUT_D = 256
N_HEADS = 8
HEAD_D = 16

LANES = 128
ROWS = 800
L_PAD = ROWS * LANES
N_TILES = 16
S_TILE = L_PAD // N_TILES
VREGS_TILE = S_TILE // 16
CAP = 1024
OUT_PAD = 1024
CHUNK = OUT_PAD // N_TILES
INF_BITS = 0x7F800000


def _thresh_body(p_ref, bits_ref, meta_ref):
    px = p_ref[0]
    py = p_ref[1]
    pz = p_ref[2]
    n = np.float32(N_ATOMS)
    c0 = jnp.sum(px) / n
    c1 = jnp.sum(py) / n
    c2 = jnp.sum(pz) / n
    dx = px - c0
    dy = py - c1
    dz = pz - c2
    d = jnp.sqrt(dx * dx + dy * dy + dz * dz)
    row = lax.broadcasted_iota(jnp.int32, (ROWS, LANES), 0)
    col = lax.broadcasted_iota(jnp.int32, (ROWS, LANES), 1)
    flat = row * LANES + col
    d = jnp.where(flat < N_ATOMS, d, jnp.float32(jnp.inf))
    bits = lax.bitcast_convert_type(d, jnp.int32)
    bits_ref[...] = bits

    def body(_, lohi):
        lo, hi = lohi
        mid = lax.div(lo + hi, jnp.int32(2))
        cnt = jnp.sum(jnp.where(bits <= mid, jnp.int32(1), jnp.int32(0)))
        pred = cnt >= K_SEL
        return (jnp.where(pred, lo, mid + 1), jnp.where(pred, mid, hi))

    _, hi = lax.fori_loop(0, 31, body, (jnp.int32(0), jnp.int32(INF_BITS)))
    t_bits = hi
    cnt_lt = jnp.sum(jnp.where(bits < t_bits, jnp.int32(1), jnp.int32(0)))
    r = jnp.int32(K_SEL) - cnt_lt
    mrow = lax.broadcasted_iota(jnp.int32, (8, LANES), 0)
    mcol = lax.broadcasted_iota(jnp.int32, (8, LANES), 1)
    mflat = mrow * LANES + mcol
    meta = jnp.where(mflat == 0, t_bits,
                     jnp.where(mflat == 1, cnt_lt,
                               jnp.where(mflat == 2, r, jnp.int32(0))))
    meta_ref[...] = meta


def _sc_body(bits_hbm, meta_hbm, x_hbm,
             xs_out, scr_out, cnt_out,
             bits_v, selbuf_v, stage_v, cnts_v, meta_v,
             idxsrc_v, idxval_v, rows_v, sem):
    cid = lax.axis_index("c")
    sid = lax.axis_index("s")
    iota = lax.iota(jnp.int32, 16)
    z16 = jnp.zeros((16,), jnp.int32)

    @pl.when(cid == 0)
    def _stage_counts():
        pltpu.sync_copy(bits_hbm.at[pl.ds(sid * S_TILE, S_TILE)], bits_v)
        pltpu.sync_copy(meta_hbm.at[pl.ds(0, 16)], meta_v)
        t_bits = jnp.sum(jnp.where(iota == 0, meta_v[...], 0))

        def cbody(j, acc):
            acc_lt, acc_eq = acc
            b = bits_v[pl.ds(j * 16, 16)]
            acc_lt = acc_lt + jnp.where(b < t_bits, 1, 0)
            acc_eq = acc_eq + jnp.where(b == t_bits, 1, 0)
            return (acc_lt, acc_eq)

        acc_lt, acc_eq = lax.fori_loop(0, VREGS_TILE, cbody, (z16, z16))
        c_lt = jnp.sum(acc_lt)
        c_eq = jnp.sum(acc_eq)
        stage_v[...] = z16 + c_lt
        pltpu.sync_copy(stage_v, cnt_out.at[pl.ds(sid * 16, 16)])
        stage_v[...] = z16 + c_eq
        pltpu.sync_copy(stage_v, cnt_out.at[pl.ds(256 + sid * 16, 16)])

    plsc.subcore_barrier()

    @pl.when(cid == 0)
    def _stage_compact():
        pltpu.sync_copy(cnt_out.at[pl.ds(0, 512)], cnts_v)
        v_lt = z16
        v_eq = z16
        for i in range(16):
            v_lt = v_lt + jnp.where(iota == i, cnts_v[pl.ds(i * 16, 16)], 0)
            v_eq = v_eq + jnp.where(iota == i, cnts_v[pl.ds(256 + i * 16, 16)], 0)
        total_lt = jnp.sum(v_lt)
        r = jnp.int32(K_SEL) - total_lt
        tie_excl = jnp.cumsum(v_eq) - v_eq
        q_vec = jnp.minimum(jnp.maximum(r - tie_excl, 0), v_eq)
        my_q = jnp.sum(jnp.where(iota == sid, q_vec, 0))
        t_bits = jnp.sum(jnp.where(iota == 0, meta_v[...], 0))

        def zbody(j, carry):
            selbuf_v[pl.ds(j * 16, 16)] = z16
            return carry

        lax.fori_loop(0, CAP // 16, zbody, 0)

        base = sid * S_TILE

        def sbody(j, carry):
            off, tcnt = carry
            b = bits_v[pl.ds(j * 16, 16)]
            m_lt = b < t_bits
            m_eq = b == t_bits
            eq1 = jnp.where(m_eq, 1, 0)
            tie_rank = tcnt + jnp.cumsum(eq1) - eq1
            m = m_lt | (m_eq & (tie_rank < my_q))
            fidx = base + j * 16 + iota
            plsc.store_compressed(selbuf_v.at[pl.ds(off, 16)], fidx, mask=m)
            off = off + jnp.sum(jnp.where(m, 1, 0))
            tcnt = tcnt + jnp.sum(eq1)
            return (off, tcnt)

        lax.fori_loop(0, VREGS_TILE, sbody, (jnp.int32(0), jnp.int32(0)))
        pltpu.sync_copy(selbuf_v, scr_out.at[pl.ds(sid * CAP, CAP)])

    plsc.subcore_barrier()

    @pl.when(cid == 0)
    def _stage_gather():
        v_lt = z16
        v_eq = z16
        for i in range(16):
            v_lt = v_lt + jnp.where(iota == i, cnts_v[pl.ds(i * 16, 16)], 0)
            v_eq = v_eq + jnp.where(iota == i, cnts_v[pl.ds(256 + i * 16, 16)], 0)
        total_lt = jnp.sum(v_lt)
        r = jnp.int32(K_SEL) - total_lt
        tie_excl = jnp.cumsum(v_eq) - v_eq
        q_vec = jnp.minimum(jnp.maximum(r - tie_excl, 0), v_eq)
        n_vec = v_lt + q_vec
        start_excl = jnp.cumsum(n_vec) - n_vec
        starts = [jnp.sum(jnp.where(iota == i, start_excl, 0)) for i in range(16)]

        p0 = sid * CHUNK
        for jj in range(CHUNK // 16):
            p = p0 + jj * 16 + iota
            tacc = z16
            for i in range(16):
                tacc = tacc + jnp.where(p >= starts[i], 1, 0)
            t_idx = tacc - 1
            ssel = z16
            for i in range(16):
                ssel = ssel + jnp.where(t_idx == i, starts[i], 0)
            src = t_idx * CAP + (p - ssel)
            idxsrc_v[pl.ds(jj * 16, 16)] = src
        pltpu.async_copy(scr_out.at[idxsrc_v], idxval_v, sem).wait()
        pltpu.async_copy(x_hbm.at[idxval_v], rows_v, sem).wait()
        pltpu.sync_copy(rows_v, xs_out.at[pl.ds(sid * CHUNK, CHUNK)])


@functools.cache
def _sc_select_gather():
    return functools.partial(
        pl.kernel,
        mesh=plsc.VectorSubcoreMesh(core_axis_name="c", subcore_axis_name="s"),
        compiler_params=pltpu.CompilerParams(needs_layout_passes=False,
                                             use_tc_tiling_on_sc=False),
        out_type=[
            jax.ShapeDtypeStruct((OUT_PAD, NODE_F), jnp.float32),
            jax.ShapeDtypeStruct((N_TILES * CAP,), jnp.int32),
            jax.ShapeDtypeStruct((512,), jnp.int32),
        ],
        scratch_types=[
            pltpu.VMEM((S_TILE,), jnp.int32),
            pltpu.VMEM((CAP,), jnp.int32),
            pltpu.VMEM((16,), jnp.int32),
            pltpu.VMEM((512,), jnp.int32),
            pltpu.VMEM((16,), jnp.int32),
            pltpu.VMEM((CHUNK,), jnp.int32),
            pltpu.VMEM((CHUNK,), jnp.int32),
            pltpu.VMEM((CHUNK, NODE_F), jnp.float32),
            pltpu.SemaphoreType.DMA,
        ],
    )(_sc_body)


def _dense_body(xs_ref, Wn_ref, bn_ref, W1_ref, b1_ref, W2_ref, b2_ref,
                Wq_ref, bq_ref, Wk_ref, bk_ref, Wv_ref, bv_ref,
                Wo_ref, bo_ref, Wp1_ref, bp1_ref, Wp2_ref, bp2_ref, out_ref):
    hp = lax.Precision.HIGHEST
    xs = xs_ref[...]
    h = jnp.dot(xs, Wn_ref[...], precision=hp) + bn_ref[...]
    h1 = jnp.maximum(jnp.dot(h, W1_ref[...], precision=hp) + b1_ref[...], 0.0)
    h = jnp.dot(h1, W2_ref[...], precision=hp) + b2_ref[...]
    q = jnp.dot(h, Wq_ref[...], precision=hp) + bq_ref[...]
    k = jnp.dot(h, Wk_ref[...], precision=hp) + bk_ref[...]
    v = jnp.dot(h, Wv_ref[...], precision=hp) + bv_ref[...]
    inv_m = np.float32(1.0 / K_SEL)
    pools = []
    for hd in range(N_HEADS):
        sl = slice(hd * HEAD_D, (hd + 1) * HEAD_D)
        qi = q[:, sl]
        ki = k[:, sl]
        vi = v[:, sl]
        s = lax.dot_general(qi, ki, (((1,), (1,)), ((), ())),
                            precision=hp) * np.float32(0.25)
        m = jnp.max(s, axis=1, keepdims=True)
        e = jnp.exp(s - m)
        z = jnp.sum(e, axis=1, keepdims=True)
        a = e / z
        w = jnp.sum(a, axis=0, keepdims=True) * inv_m
        pools.append(jnp.dot(w, vi, precision=hp))
    mo = jnp.concatenate(pools, axis=1)
    pooled = jnp.dot(mo, Wo_ref[...], precision=hp) + bo_ref[...]
    t1 = jnp.maximum(jnp.dot(pooled, Wp1_ref[...], precision=hp) + bp1_ref[...],
                     0.0)
    out_ref[...] = jnp.dot(t1, Wp2_ref[...], precision=hp) + bp2_ref[...]


def kernel(x, pos, Wn, bn, W1, b1, W2, b2, Wq, bq, Wk, bk, Wv, bv,
           Wo, bo, Wp1, bp1, Wp2, bp2):
    posT = jnp.transpose(pos)
    posP = jnp.pad(posT, ((0, 0), (0, L_PAD - N_ATOMS)))
    P = posP.reshape(3, ROWS, LANES)
    bits, meta = pl.pallas_call(
        _thresh_body,
        out_shape=[jax.ShapeDtypeStruct((ROWS, LANES), jnp.int32),
                   jax.ShapeDtypeStruct((8, LANES), jnp.int32)],
    )(P)
    xs_full, _scr, _cnt = _sc_select_gather()(
        bits.reshape(L_PAD), meta.reshape(8 * LANES), x)
    xs = xs_full[:K_SEL]
    out = pl.pallas_call(
        _dense_body,
        out_shape=jax.ShapeDtypeStruct((1, OUT_D), jnp.float32),
    )(xs, Wn, bn.reshape(1, HIDDEN), W1, b1.reshape(1, HIDDEN),
      W2, b2.reshape(1, HIDDEN), Wq, bq.reshape(1, HIDDEN),
      Wk, bk.reshape(1, HIDDEN), Wv, bv.reshape(1, HIDDEN),
      Wo, bo.reshape(1, HIDDEN), Wp1, bp1.reshape(1, OUT_D),
      Wp2, bp2.reshape(1, OUT_D))
    return out.reshape(OUT_D)


# speedup vs baseline: 1.7089x; 1.0957x over previous
import functools

import numpy as np
import jax
import jax.numpy as jnp
from jax import lax
from jax.experimental import pallas as pl
from jax.experimental.pallas import tpu as pltpu
from jax.experimental.pallas import tpu_sc as plsc

N_ATOMS = 100000
K_SEL = 1000
NODE_F = 8
HIDDEN = 128
OUT_D = 256
N_HEADS = 8
HEAD_D = 16

LANES = 128
ROWS = 800
L_PAD = ROWS * LANES
N_TILES = 16
S_TILE = L_PAD // N_TILES
VREGS_TILE = S_TILE // 16
CAP = 1024
OUT_PAD = 1024
CHUNK = OUT_PAD // N_TILES
INF_BITS = 0x7F800000


def _thresh_body(p_ref, bits_ref, meta_ref):
    px = p_ref[0]
    py = p_ref[1]
    pz = p_ref[2]
    n = np.float32(N_ATOMS)
    c0 = jnp.sum(px) / n
    c1 = jnp.sum(py) / n
    c2 = jnp.sum(pz) / n
    dx = px - c0
    dy = py - c1
    dz = pz - c2
    d = jnp.sqrt(dx * dx + dy * dy + dz * dz)
    row = lax.broadcasted_iota(jnp.int32, (ROWS, LANES), 0)
    col = lax.broadcasted_iota(jnp.int32, (ROWS, LANES), 1)
    flat = row * LANES + col
    d = jnp.where(flat < N_ATOMS, d, jnp.float32(jnp.inf))
    bits = lax.bitcast_convert_type(d, jnp.int32)
    bits_ref[...] = bits

    def body(_, lohi):
        lo, hi = lohi
        mid = lax.div(lo + hi, jnp.int32(2))
        cnt = jnp.sum(jnp.where(bits <= mid, jnp.int32(1), jnp.int32(0)))
        pred = cnt >= K_SEL
        return (jnp.where(pred, lo, mid + 1), jnp.where(pred, mid, hi))

    _, hi = lax.fori_loop(0, 31, body, (jnp.int32(0), jnp.int32(INF_BITS)))
    t_bits = hi
    cnt_lt = jnp.sum(jnp.where(bits < t_bits, jnp.int32(1), jnp.int32(0)))
    r = jnp.int32(K_SEL) - cnt_lt
    mrow = lax.broadcasted_iota(jnp.int32, (8, LANES), 0)
    mcol = lax.broadcasted_iota(jnp.int32, (8, LANES), 1)
    mflat = mrow * LANES + mcol
    meta = jnp.where(mflat == 0, t_bits,
                     jnp.where(mflat == 1, cnt_lt,
                               jnp.where(mflat == 2, r, jnp.int32(0))))
    meta_ref[...] = meta


def _sc_body(bits_hbm, meta_hbm, x_hbm,
             xs_out, scr_out, cnt_out,
             bits_v, selbuf_v, stage_v, cnts_v, meta_v,
             idxsrc_v, idxval_v, idxg_v, rows_v, sem):
    cid = lax.axis_index("c")
    sid = lax.axis_index("s")
    iota = lax.iota(jnp.int32, 16)
    z16 = jnp.zeros((16,), jnp.int32)

    @pl.when(cid == 0)
    def _stage_counts():
        pltpu.sync_copy(bits_hbm.at[pl.ds(sid * S_TILE, S_TILE)], bits_v)
        pltpu.sync_copy(meta_hbm.at[pl.ds(0, 16)], meta_v)
        t_bits = jnp.sum(jnp.where(iota == 0, meta_v[...], 0))

        def cbody(j, acc):
            acc_lt, acc_eq = acc
            b = bits_v[pl.ds(j * 16, 16)]
            acc_lt = acc_lt + jnp.where(b < t_bits, 1, 0)
            acc_eq = acc_eq + jnp.where(b == t_bits, 1, 0)
            return (acc_lt, acc_eq)

        acc_lt, acc_eq = lax.fori_loop(0, VREGS_TILE, cbody, (z16, z16))
        c_lt = jnp.sum(acc_lt)
        c_eq = jnp.sum(acc_eq)
        stage_v[...] = z16 + c_lt
        pltpu.sync_copy(stage_v, cnt_out.at[pl.ds(sid * 16, 16)])
        stage_v[...] = z16 + c_eq
        pltpu.sync_copy(stage_v, cnt_out.at[pl.ds(256 + sid * 16, 16)])

    plsc.subcore_barrier()

    @pl.when(cid == 0)
    def _stage_compact():
        pltpu.sync_copy(cnt_out.at[pl.ds(0, 512)], cnts_v)
        v_lt = z16
        v_eq = z16
        for i in range(16):
            v_lt = v_lt + jnp.where(iota == i, cnts_v[pl.ds(i * 16, 16)], 0)
            v_eq = v_eq + jnp.where(iota == i, cnts_v[pl.ds(256 + i * 16, 16)], 0)
        total_lt = jnp.sum(v_lt)
        r = jnp.int32(K_SEL) - total_lt
        tie_excl = jnp.cumsum(v_eq) - v_eq
        q_vec = jnp.minimum(jnp.maximum(r - tie_excl, 0), v_eq)
        my_q = jnp.sum(jnp.where(iota == sid, q_vec, 0))
        t_bits = jnp.sum(jnp.where(iota == 0, meta_v[...], 0))

        def zbody(j, carry):
            selbuf_v[pl.ds(j * 16, 16)] = z16
            return carry

        lax.fori_loop(0, CAP // 16, zbody, 0)

        base = sid * S_TILE

        def sbody(j, carry):
            off, tcnt = carry
            b = bits_v[pl.ds(j * 16, 16)]
            m_lt = b < t_bits
            m_eq = b == t_bits
            eq1 = jnp.where(m_eq, 1, 0)
            tie_rank = tcnt + jnp.cumsum(eq1) - eq1
            m = m_lt | (m_eq & (tie_rank < my_q))
            fidx = base + j * 16 + iota
            plsc.store_compressed(selbuf_v.at[pl.ds(off, 16)], fidx, mask=m)
            off = off + jnp.sum(jnp.where(m, 1, 0))
            tcnt = tcnt + jnp.sum(eq1)
            return (off, tcnt)

        lax.fori_loop(0, VREGS_TILE, sbody, (jnp.int32(0), jnp.int32(0)))
        pltpu.sync_copy(selbuf_v, scr_out.at[pl.ds(sid * CAP, CAP)])

    plsc.subcore_barrier()

    @pl.when(cid == 0)
    def _stage_gather():
        v_lt = z16
        v_eq = z16
        for i in range(16):
            v_lt = v_lt + jnp.where(iota == i, cnts_v[pl.ds(i * 16, 16)], 0)
            v_eq = v_eq + jnp.where(iota == i, cnts_v[pl.ds(256 + i * 16, 16)], 0)
        total_lt = jnp.sum(v_lt)
        r = jnp.int32(K_SEL) - total_lt
        tie_excl = jnp.cumsum(v_eq) - v_eq
        q_vec = jnp.minimum(jnp.maximum(r - tie_excl, 0), v_eq)
        n_vec = v_lt + q_vec
        start_excl = jnp.cumsum(n_vec) - n_vec
        starts = [jnp.sum(jnp.where(iota == i, start_excl, 0)) for i in range(16)]

        p0 = sid * CHUNK
        for jj in range(CHUNK // 16):
            p = p0 + jj * 16 + iota
            tacc = z16
            for i in range(16):
                tacc = tacc + jnp.where(p >= starts[i], 1, 0)
            t_idx = tacc - 1
            ssel = z16
            for i in range(16):
                ssel = ssel + jnp.where(t_idx == i, starts[i], 0)
            src = t_idx * CAP + (p - ssel)
            idxsrc_v[pl.ds(jj * 16, 16)] = src
        pltpu.async_copy(scr_out.at[idxsrc_v], idxval_v, sem).wait()
        for g in range(CHUNK * NODE_F // 16):
            e = g * 16 + iota
            pvec = lax.shift_right_logical(e, 3)
            cvec = e & 7
            rowv = plsc.load_gather(idxval_v, [pvec])
            idxg_v[pl.ds(g * 16, 16)] = rowv * 8 + cvec
        pltpu.async_copy(x_hbm.at[idxg_v], rows_v, sem).wait()
        pltpu.sync_copy(rows_v, xs_out.at[pl.ds(sid * CHUNK * NODE_F,
                                               CHUNK * NODE_F)])


@functools.cache
def _sc_select_gather():
    return functools.partial(
        pl.kernel,
        mesh=plsc.VectorSubcoreMesh(core_axis_name="c", subcore_axis_name="s"),
        compiler_params=pltpu.CompilerParams(needs_layout_passes=False,
                                             use_tc_tiling_on_sc=False),
        out_type=[
            jax.ShapeDtypeStruct((OUT_PAD * NODE_F,), jnp.float32),
            jax.ShapeDtypeStruct((N_TILES * CAP,), jnp.int32),
            jax.ShapeDtypeStruct((512,), jnp.int32),
        ],
        scratch_types=[
            pltpu.VMEM((S_TILE,), jnp.int32),
            pltpu.VMEM((CAP,), jnp.int32),
            pltpu.VMEM((16,), jnp.int32),
            pltpu.VMEM((512,), jnp.int32),
            pltpu.VMEM((16,), jnp.int32),
            pltpu.VMEM((CHUNK,), jnp.int32),
            pltpu.VMEM((CHUNK,), jnp.int32),
            pltpu.VMEM((CHUNK * NODE_F,), jnp.int32),
            pltpu.VMEM((CHUNK * NODE_F,), jnp.float32),
            pltpu.SemaphoreType.DMA,
        ],
    )(_sc_body)


def _dense_body(xs_ref, Wn_ref, bn_ref, W1_ref, b1_ref, W2_ref, b2_ref,
                Wq_ref, bq_ref, Wk_ref, bk_ref, Wv_ref, bv_ref,
                Wo_ref, bo_ref, Wp1_ref, bp1_ref, Wp2_ref, bp2_ref, out_ref):
    hp = lax.Precision.HIGHEST
    xs = xs_ref[...]
    h = jnp.dot(xs, Wn_ref[...], precision=hp) + bn_ref[...]
    h1 = jnp.maximum(jnp.dot(h, W1_ref[...], precision=hp) + b1_ref[...], 0.0)
    h = jnp.dot(h1, W2_ref[...], precision=hp) + b2_ref[...]
    q = jnp.dot(h, Wq_ref[...], precision=hp) + bq_ref[...]
    k = jnp.dot(h, Wk_ref[...], precision=hp) + bk_ref[...]
    v = jnp.dot(h, Wv_ref[...], precision=hp) + bv_ref[...]
    f32 = jnp.float32
    bf = jnp.bfloat16
    qs = q * np.float32(0.25)
    q_hi = qs.astype(bf)
    q_lo = (qs - q_hi.astype(f32)).astype(bf)
    k_hi = k.astype(bf)
    k_lo = (k - k_hi.astype(f32)).astype(bf)
    dn = (((1,), (1,)), ((), ()))
    inv_m = np.float32(1.0 / K_SEL)
    pools = []
    for hd in range(N_HEADS):
        sl = slice(hd * HEAD_D, (hd + 1) * HEAD_D)

        def nt(a, b, sl=sl):
            return lax.dot_general(a[:, sl], b[:, sl], dn,
                                   preferred_element_type=f32)

        s = nt(q_hi, k_hi) + nt(q_hi, k_lo) + nt(q_lo, k_hi)
        m = jnp.max(s, axis=1, keepdims=True)
        e = jnp.exp(s - m)
        z = jnp.sum(e, axis=1, keepdims=True)
        a = e * (inv_m / z)
        w = jnp.sum(a, axis=0, keepdims=True)
        pools.append(jnp.dot(w, v[:, sl], precision=hp))
    mo = jnp.concatenate(pools, axis=1)
    pooled = jnp.dot(mo, Wo_ref[...], precision=hp) + bo_ref[...]
    t1 = jnp.maximum(jnp.dot(pooled, Wp1_ref[...], precision=hp) + bp1_ref[...],
                     0.0)
    out_ref[...] = jnp.dot(t1, Wp2_ref[...], precision=hp) + bp2_ref[...]


def kernel(x, pos, Wn, bn, W1, b1, W2, b2, Wq, bq, Wk, bk, Wv, bv,
           Wo, bo, Wp1, bp1, Wp2, bp2):
    posT = jnp.transpose(pos)
    posP = jnp.pad(posT, ((0, 0), (0, L_PAD - N_ATOMS)))
    P = posP.reshape(3, ROWS, LANES)
    bits, meta = pl.pallas_call(
        _thresh_body,
        out_shape=[jax.ShapeDtypeStruct((ROWS, LANES), jnp.int32),
                   jax.ShapeDtypeStruct((8, LANES), jnp.int32)],
    )(P)
    xs_flat, _scr, _cnt = _sc_select_gather()(
        bits.reshape(L_PAD), meta.reshape(8 * LANES), x.reshape(-1))
    xs = xs_flat.reshape(OUT_PAD, NODE_F)[:K_SEL]
    out = pl.pallas_call(
        _dense_body,
        out_shape=jax.ShapeDtypeStruct((1, OUT_D), jnp.float32),
    )(xs, Wn, bn.reshape(1, HIDDEN), W1, b1.reshape(1, HIDDEN),
      W2, b2.reshape(1, HIDDEN), Wq, bq.reshape(1, HIDDEN),
      Wk, bk.reshape(1, HIDDEN), Wv, bv.reshape(1, HIDDEN),
      Wo, bo.reshape(1, HIDDEN), Wp1, bp1.reshape(1, OUT_D),
      Wp2, bp2.reshape(1, OUT_D))
    return out.reshape(OUT_D)


# speedup vs baseline: 2.2388x; 1.3101x over previous
import functools

import numpy as np
import jax
import jax.numpy as jnp
from jax import lax
from jax.experimental import pallas as pl
from jax.experimental.pallas import tpu as pltpu
from jax.experimental.pallas import tpu_sc as plsc

N_ATOMS = 100000
K_SEL = 1000
NODE_F = 8
HIDDEN = 128
OUT_D = 256
N_HEADS = 8
HEAD_D = 16

LANES = 128
ROWS = 800
L_PAD = ROWS * LANES
N_TILES = 16
S_TILE = L_PAD // N_TILES
VREGS_TILE = S_TILE // 16
CAP = 1024
OUT_PAD = 1024
CHUNK = OUT_PAD // N_TILES
INF_BITS = 0x7F800000


def _thresh_body(p_ref, bits_ref, meta_ref):
    px = p_ref[0]
    py = p_ref[1]
    pz = p_ref[2]
    n = np.float32(N_ATOMS)
    c0 = jnp.sum(px) / n
    c1 = jnp.sum(py) / n
    c2 = jnp.sum(pz) / n
    dx = px - c0
    dy = py - c1
    dz = pz - c2
    d = jnp.sqrt(dx * dx + dy * dy + dz * dz)
    row = lax.broadcasted_iota(jnp.int32, (ROWS, LANES), 0)
    col = lax.broadcasted_iota(jnp.int32, (ROWS, LANES), 1)
    flat = row * LANES + col
    d = jnp.where(flat < N_ATOMS, d, jnp.float32(jnp.inf))
    bits = lax.bitcast_convert_type(d, jnp.int32)
    bits_ref[...] = bits

    def body(_, lohi):
        lo, hi = lohi
        mid = lax.div(lo + hi, jnp.int32(2))
        cnt = jnp.sum(jnp.where(bits <= mid, jnp.int32(1), jnp.int32(0)))
        pred = cnt >= K_SEL
        return (jnp.where(pred, lo, mid + 1), jnp.where(pred, mid, hi))

    _, hi = lax.fori_loop(0, 31, body, (jnp.int32(0), jnp.int32(INF_BITS)))
    t_bits = hi
    cnt_lt = jnp.sum(jnp.where(bits < t_bits, jnp.int32(1), jnp.int32(0)))
    r = jnp.int32(K_SEL) - cnt_lt
    mrow = lax.broadcasted_iota(jnp.int32, (8, LANES), 0)
    mcol = lax.broadcasted_iota(jnp.int32, (8, LANES), 1)
    mflat = mrow * LANES + mcol
    meta = jnp.where(mflat == 0, t_bits,
                     jnp.where(mflat == 1, cnt_lt,
                               jnp.where(mflat == 2, r, jnp.int32(0))))
    meta_ref[...] = meta


def _sc_body(bits_hbm, meta_hbm, x_hbm,
             xs_out, scr_out, cnt_out,
             bits_v, selbuf_v, stage_v, cnts_v, meta_v,
             idxsrc_v, scrall_v, rows_v, sem):
    cid = lax.axis_index("c")
    sid = lax.axis_index("s")
    iota = lax.iota(jnp.int32, 16)
    z16 = jnp.zeros((16,), jnp.int32)

    @pl.when(cid == 0)
    def _stage_counts():
        pltpu.sync_copy(bits_hbm.at[pl.ds(sid * S_TILE, S_TILE)], bits_v)
        pltpu.sync_copy(meta_hbm.at[pl.ds(0, 16)], meta_v)
        t_bits = jnp.sum(jnp.where(iota == 0, meta_v[...], 0))

        def cbody(j, acc):
            acc_lt, acc_eq = acc
            b = bits_v[pl.ds(j * 16, 16)]
            acc_lt = acc_lt + jnp.where(b < t_bits, 1, 0)
            acc_eq = acc_eq + jnp.where(b == t_bits, 1, 0)
            return (acc_lt, acc_eq)

        acc_lt, acc_eq = lax.fori_loop(0, VREGS_TILE, cbody, (z16, z16))
        c_lt = jnp.sum(acc_lt)
        c_eq = jnp.sum(acc_eq)
        stage_v[...] = z16 + c_lt
        pltpu.sync_copy(stage_v, cnt_out.at[pl.ds(sid * 16, 16)])
        stage_v[...] = z16 + c_eq
        pltpu.sync_copy(stage_v, cnt_out.at[pl.ds(256 + sid * 16, 16)])

    plsc.subcore_barrier()

    @pl.when(cid == 0)
    def _stage_compact():
        pltpu.sync_copy(cnt_out.at[pl.ds(0, 512)], cnts_v)
        v_lt = z16
        v_eq = z16
        for i in range(16):
            v_lt = v_lt + jnp.where(iota == i, cnts_v[pl.ds(i * 16, 16)], 0)
            v_eq = v_eq + jnp.where(iota == i, cnts_v[pl.ds(256 + i * 16, 16)], 0)
        total_lt = jnp.sum(v_lt)
        r = jnp.int32(K_SEL) - total_lt
        tie_excl = jnp.cumsum(v_eq) - v_eq
        q_vec = jnp.minimum(jnp.maximum(r - tie_excl, 0), v_eq)
        my_q = jnp.sum(jnp.where(iota == sid, q_vec, 0))
        t_bits = jnp.sum(jnp.where(iota == 0, meta_v[...], 0))

        def zbody(j, carry):
            selbuf_v[pl.ds(j * 16, 16)] = z16
            return carry

        lax.fori_loop(0, CAP // 16, zbody, 0)

        base = sid * S_TILE

        def sbody(j, carry):
            off, tcnt = carry
            b = bits_v[pl.ds(j * 16, 16)]
            m_lt = b < t_bits
            m_eq = b == t_bits
            eq1 = jnp.where(m_eq, 1, 0)
            tie_rank = tcnt + jnp.cumsum(eq1) - eq1
            m = m_lt | (m_eq & (tie_rank < my_q))
            fidx = base + j * 16 + iota
            plsc.store_compressed(selbuf_v.at[pl.ds(off, 16)], fidx, mask=m)
            off = off + jnp.sum(jnp.where(m, 1, 0))
            tcnt = tcnt + jnp.sum(eq1)
            return (off, tcnt)

        lax.fori_loop(0, VREGS_TILE, sbody, (jnp.int32(0), jnp.int32(0)))
        pltpu.sync_copy(selbuf_v, scr_out.at[pl.ds(sid * CAP, CAP)])

    plsc.subcore_barrier()

    @pl.when(cid == 0)
    def _stage_gather():
        v_lt = z16
        v_eq = z16
        for i in range(16):
            v_lt = v_lt + jnp.where(iota == i, cnts_v[pl.ds(i * 16, 16)], 0)
            v_eq = v_eq + jnp.where(iota == i, cnts_v[pl.ds(256 + i * 16, 16)], 0)
        total_lt = jnp.sum(v_lt)
        r = jnp.int32(K_SEL) - total_lt
        tie_excl = jnp.cumsum(v_eq) - v_eq
        q_vec = jnp.minimum(jnp.maximum(r - tie_excl, 0), v_eq)
        n_vec = v_lt + q_vec
        start_excl = jnp.cumsum(n_vec) - n_vec
        starts = [jnp.sum(jnp.where(iota == i, start_excl, 0)) for i in range(16)]

        p0 = sid * CHUNK
        for jj in range(CHUNK // 16):
            p = p0 + jj * 16 + iota
            tacc = z16
            for i in range(16):
                tacc = tacc + jnp.where(p >= starts[i], 1, 0)
            t_idx = tacc - 1
            ssel = z16
            for i in range(16):
                ssel = ssel + jnp.where(t_idx == i, starts[i], 0)
            src = t_idx * CAP + (p - ssel)
            idxsrc_v[pl.ds(jj * 16, 16)] = src
        pltpu.sync_copy(scr_out, scrall_v)
        idxvals = []
        for jj in range(CHUNK // 16):
            idxvals.append(plsc.load_gather(scrall_v,
                                            [idxsrc_v[pl.ds(jj * 16, 16)]]))
        copies = []
        for j in range(CHUNK):
            vv = idxvals[j // 16]
            idx_j = jnp.sum(jnp.where(iota == (j % 16), vv, 0))
            copies.append(pltpu.make_async_copy(
                x_hbm.at[pl.ds(idx_j, 1)], rows_v.at[pl.ds(j, 1)], sem))
        for c in copies:
            c.start()
        for c in copies:
            c.wait()
        pltpu.sync_copy(rows_v, xs_out.at[pl.ds(sid * CHUNK, CHUNK)])


@functools.cache
def _sc_select_gather():
    return functools.partial(
        pl.kernel,
        mesh=plsc.VectorSubcoreMesh(core_axis_name="c", subcore_axis_name="s"),
        compiler_params=pltpu.CompilerParams(needs_layout_passes=False,
                                             use_tc_tiling_on_sc=True),
        out_type=[
            jax.ShapeDtypeStruct((OUT_PAD, NODE_F), jnp.float32),
            jax.ShapeDtypeStruct((N_TILES * CAP,), jnp.int32),
            jax.ShapeDtypeStruct((512,), jnp.int32),
        ],
        scratch_types=[
            pltpu.VMEM((S_TILE,), jnp.int32),
            pltpu.VMEM((CAP,), jnp.int32),
            pltpu.VMEM((16,), jnp.int32),
            pltpu.VMEM((512,), jnp.int32),
            pltpu.VMEM((16,), jnp.int32),
            pltpu.VMEM((CHUNK,), jnp.int32),
            pltpu.VMEM((N_TILES * CAP,), jnp.int32),
            pltpu.VMEM((CHUNK, NODE_F), jnp.float32),
            pltpu.SemaphoreType.DMA,
        ],
    )(_sc_body)


def _dense_body(xs_ref, Wn_ref, bn_ref, W1_ref, b1_ref, W2_ref, b2_ref,
                Wq_ref, bq_ref, Wk_ref, bk_ref, Wv_ref, bv_ref,
                Wo_ref, bo_ref, Wp1_ref, bp1_ref, Wp2_ref, bp2_ref, out_ref):
    hp = lax.Precision.HIGHEST
    xs = xs_ref[...]
    h = jnp.dot(xs, Wn_ref[...], precision=hp) + bn_ref[...]
    h1 = jnp.maximum(jnp.dot(h, W1_ref[...], precision=hp) + b1_ref[...], 0.0)
    h = jnp.dot(h1, W2_ref[...], precision=hp) + b2_ref[...]
    q = jnp.dot(h, Wq_ref[...], precision=hp) + bq_ref[...]
    k = jnp.dot(h, Wk_ref[...], precision=hp) + bk_ref[...]
    v = jnp.dot(h, Wv_ref[...], precision=hp) + bv_ref[...]
    f32 = jnp.float32
    bf = jnp.bfloat16
    qs = q * np.float32(0.25)
    q_hi = qs.astype(bf)
    q_lo = (qs - q_hi.astype(f32)).astype(bf)
    k_hi = k.astype(bf)
    k_lo = (k - k_hi.astype(f32)).astype(bf)
    dn = (((1,), (1,)), ((), ()))
    inv_m = np.float32(1.0 / K_SEL)
    pools = []
    for hd in range(N_HEADS):
        sl = slice(hd * HEAD_D, (hd + 1) * HEAD_D)

        def nt(a, b, sl=sl):
            return lax.dot_general(a[:, sl], b[:, sl], dn,
                                   preferred_element_type=f32)

        s = nt(q_hi, k_hi) + nt(q_hi, k_lo) + nt(q_lo, k_hi)
        m = jnp.max(s, axis=1, keepdims=True)
        e = jnp.exp(s - m)
        z = jnp.sum(e, axis=1, keepdims=True)
        a = e * (inv_m / z)
        w = jnp.sum(a, axis=0, keepdims=True)
        pools.append(jnp.dot(w, v[:, sl], precision=hp))
    mo = jnp.concatenate(pools, axis=1)
    pooled = jnp.dot(mo, Wo_ref[...], precision=hp) + bo_ref[...]
    t1 = jnp.maximum(jnp.dot(pooled, Wp1_ref[...], precision=hp) + bp1_ref[...],
                     0.0)
    out_ref[...] = jnp.dot(t1, Wp2_ref[...], precision=hp) + bp2_ref[...]


def kernel(x, pos, Wn, bn, W1, b1, W2, b2, Wq, bq, Wk, bk, Wv, bv,
           Wo, bo, Wp1, bp1, Wp2, bp2):
    posT = jnp.transpose(pos)
    posP = jnp.pad(posT, ((0, 0), (0, L_PAD - N_ATOMS)))
    P = posP.reshape(3, ROWS, LANES)
    bits, meta = pl.pallas_call(
        _thresh_body,
        out_shape=[jax.ShapeDtypeStruct((ROWS, LANES), jnp.int32),
                   jax.ShapeDtypeStruct((8, LANES), jnp.int32)],
    )(P)
    xs_full, _scr, _cnt = _sc_select_gather()(
        bits.reshape(L_PAD), meta.reshape(8 * LANES), x)
    xs = xs_full[:K_SEL]
    out = pl.pallas_call(
        _dense_body,
        out_shape=jax.ShapeDtypeStruct((1, OUT_D), jnp.float32),
    )(xs, Wn, bn.reshape(1, HIDDEN), W1, b1.reshape(1, HIDDEN),
      W2, b2.reshape(1, HIDDEN), Wq, bq.reshape(1, HIDDEN),
      Wk, bk.reshape(1, HIDDEN), Wv, bv.reshape(1, HIDDEN),
      Wo, bo.reshape(1, HIDDEN), Wp1, bp1.reshape(1, OUT_D),
      Wp2, bp2.reshape(1, OUT_D))
    return out.reshape(OUT_D)


# speedup vs baseline: 2.2765x; 1.0168x over previous
import functools

import numpy as np
import jax
import jax.numpy as jnp
from jax import lax
from jax.experimental import pallas as pl
from jax.experimental.pallas import tpu as pltpu
from jax.experimental.pallas import tpu_sc as plsc

N_ATOMS = 100000
K_SEL = 1000
NODE_F = 8
HIDDEN = 128
OUT_D = 256
N_HEADS = 8
HEAD_D = 16

LANES = 128
ROWS = 800
L_PAD = ROWS * LANES
N_TILES = 16
S_TILE = L_PAD // N_TILES
VREGS_TILE = S_TILE // 16
CAP = 1024
OUT_PAD = 1024
CHUNK = OUT_PAD // N_TILES
INF_BITS = 0x7F800000


def _thresh_body(p_ref, bits_ref, meta_ref):
    px = p_ref[0]
    py = p_ref[1]
    pz = p_ref[2]
    n = np.float32(N_ATOMS)
    c0 = jnp.sum(px) / n
    c1 = jnp.sum(py) / n
    c2 = jnp.sum(pz) / n
    dx = px - c0
    dy = py - c1
    dz = pz - c2
    d = jnp.sqrt(dx * dx + dy * dy + dz * dz)
    row = lax.broadcasted_iota(jnp.int32, (ROWS, LANES), 0)
    col = lax.broadcasted_iota(jnp.int32, (ROWS, LANES), 1)
    flat = row * LANES + col
    d = jnp.where(flat < N_ATOMS, d, jnp.float32(jnp.inf))
    bits = lax.bitcast_convert_type(d, jnp.int32)
    bits_ref[...] = bits

    def body(_, lohi):
        lo, hi = lohi
        mid = lax.div(lo + hi, jnp.int32(2))
        cnt = jnp.sum(jnp.where(bits <= mid, jnp.int32(1), jnp.int32(0)))
        pred = cnt >= K_SEL
        return (jnp.where(pred, lo, mid + 1), jnp.where(pred, mid, hi))

    _, hi = lax.fori_loop(0, 31, body, (jnp.int32(0), jnp.int32(INF_BITS)))
    t_bits = hi
    cnt_lt = jnp.sum(jnp.where(bits < t_bits, jnp.int32(1), jnp.int32(0)))
    r = jnp.int32(K_SEL) - cnt_lt
    mrow = lax.broadcasted_iota(jnp.int32, (8, LANES), 0)
    mcol = lax.broadcasted_iota(jnp.int32, (8, LANES), 1)
    mflat = mrow * LANES + mcol
    meta = jnp.where(mflat == 0, t_bits,
                     jnp.where(mflat == 1, cnt_lt,
                               jnp.where(mflat == 2, r, jnp.int32(0))))
    meta_ref[...] = meta


def _sc_body(bits_hbm, meta_hbm, x_hbm,
             xs_out, scr_out, cnt_out,
             bits_v, selbuf_v, tiebuf_v, stage_v, cnts_v, meta_v,
             idxsrc_v, idxval_v, scrall_v, rows_v, sem):
    cid = lax.axis_index("c")
    sid = lax.axis_index("s")
    iota = lax.iota(jnp.int32, 16)
    z16 = jnp.zeros((16,), jnp.int32)

    @pl.when(cid == 0)
    def _stage_select():
        pltpu.sync_copy(bits_hbm.at[pl.ds(sid * S_TILE, S_TILE)], bits_v)
        pltpu.sync_copy(meta_hbm.at[pl.ds(0, 16)], meta_v)
        t_bits = jnp.sum(jnp.where(iota == 0, meta_v[...], 0))

        def zbody(j, carry):
            selbuf_v[pl.ds(j * 16, 16)] = z16
            tiebuf_v[pl.ds(j * 16, 16)] = z16
            return carry

        lax.fori_loop(0, CAP // 16, zbody, 0)

        base = sid * S_TILE
        capv = jnp.int32(CAP - 1)

        def sbody(j, carry):
            off_lt, off_eq = carry
            b = bits_v[pl.ds(j * 16, 16)]
            m_lt = b < t_bits
            m_eq = b == t_bits
            fidx = base + j * 16 + iota
            c_lt = jnp.cumsum(jnp.where(m_lt, 1, 0))
            plsc.store_scatter(selbuf_v, [off_lt + c_lt - 1], fidx, mask=m_lt)
            c_eq = jnp.cumsum(jnp.where(m_eq, 1, 0))
            pos_eq = jnp.minimum(off_eq + c_eq - 1, capv)
            plsc.store_scatter(tiebuf_v, [pos_eq], fidx, mask=m_eq)
            off_lt = off_lt + plsc.all_reduce_population_count(m_lt)
            off_eq = off_eq + plsc.all_reduce_population_count(m_eq)
            return (off_lt, off_eq)

        off_lt, off_eq = lax.fori_loop(0, VREGS_TILE, sbody, (z16, z16))
        stage_v[...] = off_lt
        pltpu.sync_copy(stage_v, cnt_out.at[pl.ds(sid * 16, 16)])
        stage_v[...] = off_eq
        pltpu.sync_copy(stage_v, cnt_out.at[pl.ds(256 + sid * 16, 16)])

    plsc.subcore_barrier()

    @pl.when(cid == 0)
    def _stage_quota():
        pltpu.sync_copy(cnt_out.at[pl.ds(0, 512)], cnts_v)
        v_lt = z16
        v_eq = z16
        for i in range(16):
            v_lt = v_lt + jnp.where(iota == i, cnts_v[pl.ds(i * 16, 16)], 0)
            v_eq = v_eq + jnp.where(iota == i, cnts_v[pl.ds(256 + i * 16, 16)], 0)
        total_lt = jnp.sum(v_lt)
        r = jnp.int32(K_SEL) - total_lt
        tie_excl = jnp.cumsum(v_eq) - v_eq
        q_vec = jnp.minimum(jnp.maximum(r - tie_excl, 0), v_eq)
        my_q = jnp.sum(jnp.where(iota == sid, q_vec, 0))
        my_clt = jnp.sum(jnp.where(iota == sid, v_lt, 0))

        def abody(kk, carry):
            selbuf_v[pl.ds(my_clt + kk * 16, 16)] = tiebuf_v[pl.ds(kk * 16, 16)]
            return carry

        lax.fori_loop(0, (my_q + 15) // 16, abody, 0)
        pltpu.sync_copy(selbuf_v, scr_out.at[pl.ds(sid * CAP, CAP)])

    plsc.subcore_barrier()

    @pl.when(cid == 0)
    def _stage_gather():
        v_lt = z16
        v_eq = z16
        for i in range(16):
            v_lt = v_lt + jnp.where(iota == i, cnts_v[pl.ds(i * 16, 16)], 0)
            v_eq = v_eq + jnp.where(iota == i, cnts_v[pl.ds(256 + i * 16, 16)], 0)
        total_lt = jnp.sum(v_lt)
        r = jnp.int32(K_SEL) - total_lt
        tie_excl = jnp.cumsum(v_eq) - v_eq
        q_vec = jnp.minimum(jnp.maximum(r - tie_excl, 0), v_eq)
        n_vec = v_lt + q_vec
        start_excl = jnp.cumsum(n_vec) - n_vec
        starts = [jnp.sum(jnp.where(iota == i, start_excl, 0)) for i in range(16)]

        p0 = sid * CHUNK
        for jj in range(CHUNK // 16):
            p = p0 + jj * 16 + iota
            tacc = z16
            for i in range(16):
                tacc = tacc + jnp.where(p >= starts[i], 1, 0)
            t_idx = tacc - 1
            ssel = z16
            for i in range(16):
                ssel = ssel + jnp.where(t_idx == i, starts[i], 0)
            src = t_idx * CAP + (p - ssel)
            idxsrc_v[pl.ds(jj * 16, 16)] = src
        pltpu.sync_copy(scr_out, scrall_v)
        for jj in range(CHUNK // 16):
            idxval_v[pl.ds(jj * 16, 16)] = plsc.load_gather(
                scrall_v, [idxsrc_v[pl.ds(jj * 16, 16)]])

        idxvals = [idxval_v[pl.ds(jj * 16, 16)] for jj in range(CHUNK // 16)]
        copies = []
        for j in range(CHUNK):
            idx_j = jnp.sum(jnp.where(iota == (j % 16), idxvals[j // 16], 0))
            copies.append(pltpu.make_async_copy(
                x_hbm.at[pl.ds(idx_j, 1)], rows_v.at[pl.ds(j, 1)], sem))
        for c in copies:
            c.start()
        for c in copies:
            c.wait()
        pltpu.sync_copy(rows_v, xs_out.at[pl.ds(sid * CHUNK, CHUNK)])


@functools.cache
def _sc_select_gather():
    return functools.partial(
        pl.kernel,
        mesh=plsc.VectorSubcoreMesh(core_axis_name="c", subcore_axis_name="s"),
        compiler_params=pltpu.CompilerParams(needs_layout_passes=False,
                                             use_tc_tiling_on_sc=True),
        out_type=[
            jax.ShapeDtypeStruct((OUT_PAD, NODE_F), jnp.float32),
            jax.ShapeDtypeStruct((N_TILES * CAP,), jnp.int32),
            jax.ShapeDtypeStruct((2048,), jnp.int32),
        ],
        scratch_types=[
            pltpu.VMEM((S_TILE,), jnp.int32),
            pltpu.VMEM((CAP,), jnp.int32),
            pltpu.VMEM((CAP,), jnp.int32),
            pltpu.VMEM((16,), jnp.int32),
            pltpu.VMEM((512,), jnp.int32),
            pltpu.VMEM((16,), jnp.int32),
            pltpu.VMEM((CHUNK,), jnp.int32),
            pltpu.VMEM((CHUNK,), jnp.int32),
            pltpu.VMEM((N_TILES * CAP,), jnp.int32),
            pltpu.VMEM((CHUNK, NODE_F), jnp.float32),
            pltpu.SemaphoreType.DMA,
        ],
    )(_sc_body)


def _dense_body(xs_ref, Wn_ref, bn_ref, W1_ref, b1_ref, W2_ref, b2_ref,
                Wq_ref, bq_ref, Wk_ref, bk_ref, Wv_ref, bv_ref,
                Wo_ref, bo_ref, Wp1_ref, bp1_ref, Wp2_ref, bp2_ref, out_ref):
    hp = lax.Precision.HIGHEST
    xs = xs_ref[...]
    h = jnp.dot(xs, Wn_ref[...], precision=hp) + bn_ref[...]
    h1 = jnp.maximum(jnp.dot(h, W1_ref[...], precision=hp) + b1_ref[...], 0.0)
    h = jnp.dot(h1, W2_ref[...], precision=hp) + b2_ref[...]
    q = jnp.dot(h, Wq_ref[...], precision=hp) + bq_ref[...]
    k = jnp.dot(h, Wk_ref[...], precision=hp) + bk_ref[...]
    v = jnp.dot(h, Wv_ref[...], precision=hp) + bv_ref[...]
    f32 = jnp.float32
    bf = jnp.bfloat16
    qs = q * np.float32(0.25)
    q_hi = qs.astype(bf)
    q_lo = (qs - q_hi.astype(f32)).astype(bf)
    k_hi = k.astype(bf)
    k_lo = (k - k_hi.astype(f32)).astype(bf)
    dn = (((1,), (1,)), ((), ()))
    inv_m = np.float32(1.0 / K_SEL)
    pools = []
    for hd in range(N_HEADS):
        sl = slice(hd * HEAD_D, (hd + 1) * HEAD_D)

        def nt(a, b, sl=sl):
            return lax.dot_general(a[:, sl], b[:, sl], dn,
                                   preferred_element_type=f32)

        s = nt(q_hi, k_hi) + nt(q_hi, k_lo) + nt(q_lo, k_hi)
        m = jnp.max(s, axis=1, keepdims=True)
        e = jnp.exp(s - m)
        z = jnp.sum(e, axis=1, keepdims=True)
        a = e * (inv_m / z)
        w = jnp.sum(a, axis=0, keepdims=True)
        pools.append(jnp.dot(w, v[:, sl], precision=hp))
    mo = jnp.concatenate(pools, axis=1)
    pooled = jnp.dot(mo, Wo_ref[...], precision=hp) + bo_ref[...]
    t1 = jnp.maximum(jnp.dot(pooled, Wp1_ref[...], precision=hp) + bp1_ref[...],
                     0.0)
    out_ref[...] = jnp.dot(t1, Wp2_ref[...], precision=hp) + bp2_ref[...]


def kernel(x, pos, Wn, bn, W1, b1, W2, b2, Wq, bq, Wk, bk, Wv, bv,
           Wo, bo, Wp1, bp1, Wp2, bp2):
    posT = jnp.transpose(pos)
    posP = jnp.pad(posT, ((0, 0), (0, L_PAD - N_ATOMS)))
    P = posP.reshape(3, ROWS, LANES)
    bits, meta = pl.pallas_call(
        _thresh_body,
        out_shape=[jax.ShapeDtypeStruct((ROWS, LANES), jnp.int32),
                   jax.ShapeDtypeStruct((8, LANES), jnp.int32)],
    )(P)
    xs_full, _scr, _cnt = _sc_select_gather()(
        bits.reshape(L_PAD), meta.reshape(8 * LANES), x)
    xs = xs_full[:K_SEL]
    out = pl.pallas_call(
        _dense_body,
        out_shape=jax.ShapeDtypeStruct((1, OUT_D), jnp.float32),
    )(xs, Wn, bn.reshape(1, HIDDEN), W1, b1.reshape(1, HIDDEN),
      W2, b2.reshape(1, HIDDEN), Wq, bq.reshape(1, HIDDEN),
      Wk, bk.reshape(1, HIDDEN), Wv, bv.reshape(1, HIDDEN),
      Wo, bo.reshape(1, HIDDEN), Wp1, bp1.reshape(1, OUT_D),
      Wp2, bp2.reshape(1, OUT_D))
    return out.reshape(OUT_D)


# speedup vs baseline: 2.4244x; 1.0650x over previous
import functools

import numpy as np
import jax
import jax.numpy as jnp
from jax import lax
from jax.experimental import pallas as pl
from jax.experimental.pallas import tpu as pltpu
from jax.experimental.pallas import tpu_sc as plsc

N_ATOMS = 100000
K_SEL = 1000
NODE_F = 8
HIDDEN = 128
OUT_D = 256
N_HEADS = 8
HEAD_D = 16

LANES = 128
ROWS = 800
L_PAD = ROWS * LANES
N_TILES = 16
S_TILE = L_PAD // N_TILES
VREGS_TILE = S_TILE // 16
CAP = 1024
OUT_PAD = 1024
CHUNK = OUT_PAD // N_TILES
INF_BITS = 0x7F800000


def _thresh_body(p_ref, bits_ref, meta_ref):
    px = p_ref[0]
    py = p_ref[1]
    pz = p_ref[2]
    n = np.float32(N_ATOMS)
    c0 = jnp.sum(px) / n
    c1 = jnp.sum(py) / n
    c2 = jnp.sum(pz) / n
    dx = px - c0
    dy = py - c1
    dz = pz - c2
    d = jnp.sqrt(dx * dx + dy * dy + dz * dz)
    row = lax.broadcasted_iota(jnp.int32, (ROWS, LANES), 0)
    col = lax.broadcasted_iota(jnp.int32, (ROWS, LANES), 1)
    flat = row * LANES + col
    d = jnp.where(flat < N_ATOMS, d, jnp.float32(jnp.inf))
    bits = lax.bitcast_convert_type(d, jnp.int32)
    bits_ref[...] = bits

    def body(_, lohi):
        lo, hi = lohi
        mid = lax.div(lo + hi, jnp.int32(2))
        cnt = jnp.sum(jnp.where(bits <= mid, jnp.int32(1), jnp.int32(0)))
        pred = cnt >= K_SEL
        return (jnp.where(pred, lo, mid + 1), jnp.where(pred, mid, hi))

    _, hi = lax.fori_loop(0, 31, body, (jnp.int32(0), jnp.int32(INF_BITS)))
    t_bits = hi
    cnt_lt = jnp.sum(jnp.where(bits < t_bits, jnp.int32(1), jnp.int32(0)))
    r = jnp.int32(K_SEL) - cnt_lt
    mrow = lax.broadcasted_iota(jnp.int32, (8, LANES), 0)
    mcol = lax.broadcasted_iota(jnp.int32, (8, LANES), 1)
    mflat = mrow * LANES + mcol
    meta = jnp.where(mflat == 0, t_bits,
                     jnp.where(mflat == 1, cnt_lt,
                               jnp.where(mflat == 2, r, jnp.int32(0))))
    meta_ref[...] = meta


def _sc_body(bits_hbm, meta_hbm, x_hbm,
             xs_out, scr_out, cnt_out,
             bits_v, selbuf_v, tiebuf_v, stage_v, cnts_v, meta_v,
             idxsrc_v, idxval_v, scrall_v, rows_v, sem):
    cid = lax.axis_index("c")
    sid = lax.axis_index("s")
    iota = lax.iota(jnp.int32, 16)
    z16 = jnp.zeros((16,), jnp.int32)

    @pl.when(cid == 0)
    def _stage_select():
        pltpu.sync_copy(bits_hbm.at[pl.ds(sid * S_TILE, S_TILE)], bits_v)
        pltpu.sync_copy(meta_hbm.at[pl.ds(0, 16)], meta_v)
        t_bits = jnp.sum(jnp.where(iota == 0, meta_v[...], 0))

        def zbody(j, carry):
            selbuf_v[pl.ds(j * 16, 16)] = z16
            tiebuf_v[pl.ds(j * 16, 16)] = z16
            return carry

        lax.fori_loop(0, CAP // 16, zbody, 0)

        base = sid * S_TILE
        capv = jnp.int32(CAP - 1)

        def sbody(j, carry):
            off_lt, off_eq = carry
            b = bits_v[pl.ds(j * 16, 16)]
            m_lt = b < t_bits
            m_eq = b == t_bits
            fidx = base + j * 16 + iota
            c_lt = jnp.cumsum(jnp.where(m_lt, 1, 0))
            plsc.store_scatter(selbuf_v, [off_lt + c_lt - 1], fidx, mask=m_lt)
            c_eq = jnp.cumsum(jnp.where(m_eq, 1, 0))
            pos_eq = jnp.minimum(off_eq + c_eq - 1, capv)
            plsc.store_scatter(tiebuf_v, [pos_eq], fidx, mask=m_eq)
            off_lt = off_lt + plsc.all_reduce_population_count(m_lt)
            off_eq = off_eq + plsc.all_reduce_population_count(m_eq)
            return (off_lt, off_eq)

        off_lt, off_eq = lax.fori_loop(0, VREGS_TILE, sbody, (z16, z16))
        stage_v[...] = off_lt
        pltpu.sync_copy(stage_v, cnt_out.at[pl.ds(sid * 16, 16)])
        stage_v[...] = off_eq
        pltpu.sync_copy(stage_v, cnt_out.at[pl.ds(256 + sid * 16, 16)])

    plsc.subcore_barrier()

    @pl.when(cid == 0)
    def _stage_quota():
        pltpu.sync_copy(cnt_out.at[pl.ds(0, 512)], cnts_v)
        v_lt = z16
        v_eq = z16
        for i in range(16):
            v_lt = v_lt + jnp.where(iota == i, cnts_v[pl.ds(i * 16, 16)], 0)
            v_eq = v_eq + jnp.where(iota == i, cnts_v[pl.ds(256 + i * 16, 16)], 0)
        total_lt = jnp.sum(v_lt)
        r = jnp.int32(K_SEL) - total_lt
        tie_excl = jnp.cumsum(v_eq) - v_eq
        q_vec = jnp.minimum(jnp.maximum(r - tie_excl, 0), v_eq)
        my_q = jnp.sum(jnp.where(iota == sid, q_vec, 0))
        my_clt = jnp.sum(jnp.where(iota == sid, v_lt, 0))

        def abody(kk, carry):
            selbuf_v[pl.ds(my_clt + kk * 16, 16)] = tiebuf_v[pl.ds(kk * 16, 16)]
            return carry

        lax.fori_loop(0, (my_q + 15) // 16, abody, 0)
        pltpu.sync_copy(selbuf_v, scr_out.at[pl.ds(sid * CAP, CAP)])

    plsc.subcore_barrier()

    @pl.when(cid == 0)
    def _stage_gather():
        v_lt = z16
        v_eq = z16
        for i in range(16):
            v_lt = v_lt + jnp.where(iota == i, cnts_v[pl.ds(i * 16, 16)], 0)
            v_eq = v_eq + jnp.where(iota == i, cnts_v[pl.ds(256 + i * 16, 16)], 0)
        total_lt = jnp.sum(v_lt)
        r = jnp.int32(K_SEL) - total_lt
        tie_excl = jnp.cumsum(v_eq) - v_eq
        q_vec = jnp.minimum(jnp.maximum(r - tie_excl, 0), v_eq)
        n_vec = v_lt + q_vec
        start_excl = jnp.cumsum(n_vec) - n_vec
        starts = [jnp.sum(jnp.where(iota == i, start_excl, 0)) for i in range(16)]

        p0 = sid * CHUNK
        for jj in range(CHUNK // 16):
            p = p0 + jj * 16 + iota
            tacc = z16
            for i in range(16):
                tacc = tacc + jnp.where(p >= starts[i], 1, 0)
            t_idx = tacc - 1
            ssel = z16
            for i in range(16):
                ssel = ssel + jnp.where(t_idx == i, starts[i], 0)
            src = t_idx * CAP + (p - ssel)
            idxsrc_v[pl.ds(jj * 16, 16)] = src
        pltpu.sync_copy(scr_out, scrall_v)
        for jj in range(CHUNK // 16):
            idxval_v[pl.ds(jj * 16, 16)] = plsc.load_gather(
                scrall_v, [idxsrc_v[pl.ds(jj * 16, 16)]])

        idxvals = [idxval_v[pl.ds(jj * 16, 16)] for jj in range(CHUNK // 16)]
        copies = []
        for j in range(CHUNK):
            idx_j = jnp.sum(jnp.where(iota == (j % 16), idxvals[j // 16], 0))
            copies.append(pltpu.make_async_copy(
                x_hbm.at[pl.ds(idx_j, 1)], rows_v.at[pl.ds(j, 1)], sem))
        for c in copies:
            c.start()
        for c in copies:
            c.wait()
        pltpu.sync_copy(rows_v, xs_out.at[pl.ds(sid * CHUNK, CHUNK)])


@functools.cache
def _sc_select_gather():
    return functools.partial(
        pl.kernel,
        mesh=plsc.VectorSubcoreMesh(core_axis_name="c", subcore_axis_name="s"),
        compiler_params=pltpu.CompilerParams(needs_layout_passes=False,
                                             use_tc_tiling_on_sc=True),
        out_type=[
            jax.ShapeDtypeStruct((OUT_PAD, NODE_F), jnp.float32),
            jax.ShapeDtypeStruct((N_TILES * CAP,), jnp.int32),
            jax.ShapeDtypeStruct((2048,), jnp.int32),
        ],
        scratch_types=[
            pltpu.VMEM((S_TILE,), jnp.int32),
            pltpu.VMEM((CAP,), jnp.int32),
            pltpu.VMEM((CAP,), jnp.int32),
            pltpu.VMEM((16,), jnp.int32),
            pltpu.VMEM((512,), jnp.int32),
            pltpu.VMEM((16,), jnp.int32),
            pltpu.VMEM((CHUNK,), jnp.int32),
            pltpu.VMEM((CHUNK,), jnp.int32),
            pltpu.VMEM((N_TILES * CAP,), jnp.int32),
            pltpu.VMEM((CHUNK, NODE_F), jnp.float32),
            pltpu.SemaphoreType.DMA,
        ],
    )(_sc_body)


def _dense_body(xs_ref, Wn_ref, bn_ref, W1_ref, b1_ref, W2_ref, b2_ref,
                Wq_ref, bq_ref, Wk_ref, bk_ref, Wv_ref, bv_ref,
                Wo_ref, bo_ref, Wp1_ref, bp1_ref, Wp2_ref, bp2_ref, out_ref):
    hp = lax.Precision.HIGHEST
    xs = xs_ref[...]
    h = jnp.dot(xs, Wn_ref[...], precision=hp) + bn_ref[...]
    h1 = jnp.maximum(jnp.dot(h, W1_ref[...], precision=hp) + b1_ref[...], 0.0)
    h = jnp.dot(h1, W2_ref[...], precision=hp) + b2_ref[...]
    q = jnp.dot(h, Wq_ref[...], precision=hp) + bq_ref[...]
    k = jnp.dot(h, Wk_ref[...], precision=hp) + bk_ref[...]
    v = jnp.dot(h, Wv_ref[...], precision=hp) + bv_ref[...]
    f32 = jnp.float32
    bf = jnp.bfloat16
    qs = q * np.float32(0.25)
    q_hi = qs.astype(bf)
    q_lo = (qs - q_hi.astype(f32)).astype(bf)
    k_hi = k.astype(bf)
    k_lo = (k - k_hi.astype(f32)).astype(bf)
    dn = (((1,), (1,)), ((), ()))
    inv_m = np.float32(1.0 / K_SEL)
    colmask = (lax.broadcasted_iota(jnp.int32, (1, OUT_PAD), 1)
               < K_SEL).astype(f32)
    rowvalid = lax.broadcasted_iota(jnp.int32, (OUT_PAD, 1), 0) < K_SEL
    pools = []
    for hd in range(N_HEADS):
        sl = slice(hd * HEAD_D, (hd + 1) * HEAD_D)

        def nt(a, b, sl=sl):
            return lax.dot_general(a[:, sl], b[:, sl], dn,
                                   preferred_element_type=f32)

        s = nt(q_hi, k_hi) + nt(q_hi, k_lo) + nt(q_lo, k_hi)
        m = jnp.max(s, axis=1, keepdims=True)
        e = jnp.exp(s - m) * colmask
        z = jnp.sum(e, axis=1, keepdims=True)
        a = e * jnp.where(rowvalid, inv_m / z, 0.0)
        w = jnp.sum(a, axis=0, keepdims=True)
        pools.append(jnp.dot(w, v[:, sl], precision=hp))
    mo = jnp.concatenate(pools, axis=1)
    pooled = jnp.dot(mo, Wo_ref[...], precision=hp) + bo_ref[...]
    t1 = jnp.maximum(jnp.dot(pooled, Wp1_ref[...], precision=hp) + bp1_ref[...],
                     0.0)
    out_ref[...] = jnp.dot(t1, Wp2_ref[...], precision=hp) + bp2_ref[...]


def kernel(x, pos, Wn, bn, W1, b1, W2, b2, Wq, bq, Wk, bk, Wv, bv,
           Wo, bo, Wp1, bp1, Wp2, bp2):
    posT = jnp.transpose(pos)
    posP = jnp.pad(posT, ((0, 0), (0, L_PAD - N_ATOMS)))
    P = posP.reshape(3, ROWS, LANES)
    bits, meta = pl.pallas_call(
        _thresh_body,
        out_shape=[jax.ShapeDtypeStruct((ROWS, LANES), jnp.int32),
                   jax.ShapeDtypeStruct((8, LANES), jnp.int32)],
    )(P)
    xs_full, _scr, _cnt = _sc_select_gather()(
        bits.reshape(L_PAD), meta.reshape(8 * LANES), x)
    out = pl.pallas_call(
        _dense_body,
        out_shape=jax.ShapeDtypeStruct((1, OUT_D), jnp.float32),
    )(xs_full, Wn, bn.reshape(1, HIDDEN), W1, b1.reshape(1, HIDDEN),
      W2, b2.reshape(1, HIDDEN), Wq, bq.reshape(1, HIDDEN),
      Wk, bk.reshape(1, HIDDEN), Wv, bv.reshape(1, HIDDEN),
      Wo, bo.reshape(1, HIDDEN), Wp1, bp1.reshape(1, OUT_D),
      Wp2, bp2.reshape(1, OUT_D))
    return out.reshape(OUT_D)
